# Initial kernel scaffold; baseline (speedup 1.0000x reference)
import jax
import jax.numpy as jnp
from jax.experimental import pallas as pl


def kernel(user_emb, item_emb, edge_vals, edge_index):
    raise NotImplementedError("write your pallas kernel here")


# speedup vs baseline: 2.4620x; 2.4620x over previous
import functools

import jax
import jax.numpy as jnp
from jax import lax
from jax.experimental import pallas as pl
from jax.experimental.pallas import tpu as pltpu, tpu_sc as plsc

NUM_USER = 25000
NUM_ITEM = 25000
N_NODES = NUM_USER + NUM_ITEM
EMBED_DIM = 64
N_EDGES = 800000
N_LAYERS = 4

NC, NS = 2, 16
NW = NC * NS
HALF = N_NODES // NC
GARBAGE = HALF
ACC_ROWS = HALF + 8

CHUNK = 128
E_PAD = ((N_EDGES + NS * CHUNK - 1) // (NS * CHUNK)) * (NS * CHUNK)
CHUNKS_PER_TILE = E_PAD // (NS * CHUNK)
EDGES_PER_TILE = CHUNKS_PER_TILE * CHUNK

OUT_ROWS = N_NODES + 16

COPY_ROWS = 200
N_COPY = HALF // COPY_ROWS


def _layer_body(embeds_hbm, src_hbm, dst_hbm, val_hbm, zeros_hbm, out_hbm,
                acc, srcv, dstv, valv, rows_v):
    core = lax.axis_index("c")
    tile = lax.axis_index("s")
    out_base = core * HALF

    for k in range(8):
        cidx = tile + NS * k
        @pl.when(cidx < N_COPY)
        def _():
            pltpu.sync_copy(zeros_hbm.at[pl.ds(cidx * COPY_ROWS, COPY_ROWS)],
                            acc.at[pl.ds(cidx * COPY_ROWS, COPY_ROWS)])
    plsc.subcore_barrier()

    edge_base = tile * EDGES_PER_TILE

    def chunk_body(i, carry):
        base = edge_base + i * CHUNK
        pltpu.sync_copy(src_hbm.at[pl.ds(base, CHUNK)], srcv)
        pltpu.sync_copy(dst_hbm.at[pl.ds(base, CHUNK)], dstv)
        pltpu.sync_copy(val_hbm.at[pl.ds(base, CHUNK)], valv)
        pltpu.sync_copy(embeds_hbm.at[srcv], rows_v)
        for g in range(CHUNK // 16):
            d16 = dstv[pl.ds(g * 16, 16)] - core * HALF
            ok = (d16 >= 0) & (d16 < HALF)
            d16 = jnp.where(ok, d16, GARBAGE)
            v16 = valv[pl.ds(g * 16, 16)]
            for e in range(16):
                ge = g * 16 + e
                v = v16[e]
                for j in range(EMBED_DIM // 16):
                    col = pl.ds(j * 16, 16)
                    rows_v[ge, col] = rows_v[ge, col] * v
            pltpu.sync_copy(rows_v.at[pl.ds(g * 16, 16)], acc.at[d16],
                            add=True)
        return carry

    lax.fori_loop(0, CHUNKS_PER_TILE, chunk_body, 0)
    plsc.subcore_barrier()

    for k in range(8):
        cidx = tile + NS * k
        @pl.when(cidx < N_COPY)
        def _():
            pltpu.sync_copy(
                acc.at[pl.ds(cidx * COPY_ROWS, COPY_ROWS)],
                out_hbm.at[pl.ds(out_base + cidx * COPY_ROWS, COPY_ROWS)])


@jax.jit
def _run_layer(embeds, src, dst, vals, zeros):
    mesh = plsc.VectorSubcoreMesh(core_axis_name="c", subcore_axis_name="s")
    f = pl.kernel(
        _layer_body,
        out_type=jax.ShapeDtypeStruct((OUT_ROWS, EMBED_DIM), jnp.float32),
        mesh=mesh,
        scratch_types=[
            pltpu.VMEM_SHARED((ACC_ROWS, EMBED_DIM), jnp.float32),
            pltpu.VMEM((CHUNK,), jnp.int32),
            pltpu.VMEM((CHUNK,), jnp.int32),
            pltpu.VMEM((CHUNK,), jnp.float32),
            pltpu.VMEM((CHUNK, EMBED_DIM), jnp.float32),
        ],
        compiler_params=pltpu.CompilerParams(use_tc_tiling_on_sc=False),
    )
    return f(embeds, src, dst, vals, zeros)


def kernel(user_emb, item_emb, edge_vals, edge_index):
    embeds = jnp.concatenate([user_emb, item_emb], axis=0)
    embeds = jnp.pad(embeds, ((0, OUT_ROWS - N_NODES), (0, 0)))
    pad_e = E_PAD - N_EDGES
    src = jnp.pad(edge_index[0], (0, pad_e))
    dst = jnp.pad(edge_index[1], (0, pad_e), constant_values=N_NODES + 1)
    vals = jnp.pad(edge_vals, (0, pad_e))
    zeros = jnp.zeros((HALF, EMBED_DIM), jnp.float32)
    for _ in range(N_LAYERS):
        embeds = _run_layer(embeds, src, dst, vals, zeros)
    return embeds[:NUM_USER], embeds[NUM_USER:N_NODES]


# speedup vs baseline: 5.1511x; 2.0923x over previous
import jax
import jax.numpy as jnp
from jax import lax
from jax.experimental import pallas as pl
from jax.experimental.pallas import tpu as pltpu, tpu_sc as plsc

NUM_USER = 25000
NUM_ITEM = 25000
N_NODES = NUM_USER + NUM_ITEM
EMBED_DIM = 64
N_EDGES = 800000
N_LAYERS = 4

NC, NS = 2, 16
HALF = N_NODES // NC
GARBAGE = HALF
ACC_ROWS = HALF + 8

CHUNK = 128
CHUNKS_PER_TILE = 392
PAIRS = CHUNKS_PER_TILE // 2
BLOCK_CHUNKS = 28
BLOCK_E = BLOCK_CHUNKS * CHUNK
EDGES_PER_TILE = CHUNKS_PER_TILE * CHUNK
E_PAD = NS * EDGES_PER_TILE

OUT_ROWS = N_NODES + 16

COPY_ROWS = 200
N_COPY = HALF // COPY_ROWS


def _scale_and_scatter(rows, dstb, valb, l0, core, acc, sem):
    for g in range(CHUNK // 16):
        off = l0 + g * 16
        d16 = dstb[pl.ds(off, 16)] - core * HALF
        ok = (d16 >= 0) & (d16 < HALF)
        d16 = jnp.where(ok, d16, GARBAGE)
        v16 = valb[pl.ds(off, 16)]
        for e in range(16):
            ge = g * 16 + e
            v = v16[e]
            for j in range(EMBED_DIM // 16):
                col = pl.ds(j * 16, 16)
                rows[ge, col] = rows[ge, col] * v
        pltpu.async_copy(rows.at[pl.ds(g * 16, 16)], acc.at[d16], sem,
                         add=True)


def _layer_body(embeds_hbm, src_hbm, dst_hbm, val_hbm, zeros_hbm, out_hbm,
                acc, srcb, dstb, valb, rows0, rows1, g0, g1, s0, s1):
    core = lax.axis_index("c")
    tile = lax.axis_index("s")
    out_base = core * HALF

    for k in range(8):
        cidx = tile + NS * k
        @pl.when(cidx < N_COPY)
        def _():
            pltpu.sync_copy(zeros_hbm.at[pl.ds(cidx * COPY_ROWS, COPY_ROWS)],
                            acc.at[pl.ds(cidx * COPY_ROWS, COPY_ROWS)])
    plsc.subcore_barrier()

    edge_base = tile * EDGES_PER_TILE
    dummy = zeros_hbm.at[pl.ds(0, CHUNK)]

    def load_block(nb):
        base = edge_base + nb * BLOCK_E
        pltpu.sync_copy(src_hbm.at[pl.ds(base, BLOCK_E)], srcb)
        pltpu.sync_copy(dst_hbm.at[pl.ds(base, BLOCK_E)], dstb)
        pltpu.sync_copy(val_hbm.at[pl.ds(base, BLOCK_E)], valb)

    def drain(sem):
        pltpu.make_async_copy(dummy, rows0, sem).wait()

    load_block(0)
    pltpu.async_copy(embeds_hbm.at[srcb.at[pl.ds(0, CHUNK)]], rows0, g0)

    def pair_body(i, carry):
        l0 = lax.rem(2 * i, BLOCK_CHUNKS) * CHUNK
        l1 = l0 + CHUNK

        @pl.when(i > 0)
        def _():
            drain(s1)
        pltpu.async_copy(embeds_hbm.at[srcb.at[pl.ds(l1, CHUNK)]], rows1, g1)
        drain(g0)
        _scale_and_scatter(rows0, dstb, valb, l0, core, acc, s0)
        drain(g1)
        drain(s0)

        boundary = lax.rem(2 * i + 2, BLOCK_CHUNKS) == 0

        @pl.when(jnp.logical_and(jnp.logical_not(boundary), i < PAIRS - 1))
        def _():
            l0n = lax.rem(2 * i + 2, BLOCK_CHUNKS) * CHUNK
            pltpu.async_copy(embeds_hbm.at[srcb.at[pl.ds(l0n, CHUNK)]],
                             rows0, g0)

        _scale_and_scatter(rows1, dstb, valb, l1, core, acc, s1)

        @pl.when(jnp.logical_and(boundary, i < PAIRS - 1))
        def _():
            load_block((2 * i + 2) // BLOCK_CHUNKS)
            pltpu.async_copy(embeds_hbm.at[srcb.at[pl.ds(0, CHUNK)]],
                             rows0, g0)
        return carry

    lax.fori_loop(0, PAIRS, pair_body, 0)
    drain(s1)
    plsc.subcore_barrier()

    for k in range(8):
        cidx = tile + NS * k
        @pl.when(cidx < N_COPY)
        def _():
            pltpu.sync_copy(
                acc.at[pl.ds(cidx * COPY_ROWS, COPY_ROWS)],
                out_hbm.at[pl.ds(out_base + cidx * COPY_ROWS, COPY_ROWS)])


@jax.jit
def _run_layer(embeds, src, dst, vals, zeros):
    mesh = plsc.VectorSubcoreMesh(core_axis_name="c", subcore_axis_name="s")
    f = pl.kernel(
        _layer_body,
        out_type=jax.ShapeDtypeStruct((OUT_ROWS, EMBED_DIM), jnp.float32),
        mesh=mesh,
        scratch_types=[
            pltpu.VMEM_SHARED((ACC_ROWS, EMBED_DIM), jnp.float32),
            pltpu.VMEM((BLOCK_E,), jnp.int32),
            pltpu.VMEM((BLOCK_E,), jnp.int32),
            pltpu.VMEM((BLOCK_E,), jnp.float32),
            pltpu.VMEM((CHUNK, EMBED_DIM), jnp.float32),
            pltpu.VMEM((CHUNK, EMBED_DIM), jnp.float32),
            pltpu.SemaphoreType.DMA,
            pltpu.SemaphoreType.DMA,
            pltpu.SemaphoreType.DMA,
            pltpu.SemaphoreType.DMA,
        ],
        compiler_params=pltpu.CompilerParams(use_tc_tiling_on_sc=False),
    )
    return f(embeds, src, dst, vals, zeros)


def kernel(user_emb, item_emb, edge_vals, edge_index):
    embeds = jnp.concatenate([user_emb, item_emb], axis=0)
    embeds = jnp.pad(embeds, ((0, OUT_ROWS - N_NODES), (0, 0)))
    pad_e = E_PAD - N_EDGES
    src = jnp.pad(edge_index[0], (0, pad_e))
    dst = jnp.pad(edge_index[1], (0, pad_e), constant_values=N_NODES + 1)
    vals = jnp.pad(edge_vals, (0, pad_e))
    zeros = jnp.zeros((HALF, EMBED_DIM), jnp.float32)
    for _ in range(N_LAYERS):
        embeds = _run_layer(embeds, src, dst, vals, zeros)
    return embeds[:NUM_USER], embeds[NUM_USER:N_NODES]


# speedup vs baseline: 6.1817x; 1.2001x over previous
import jax
import jax.numpy as jnp
from jax import lax
from jax.experimental import pallas as pl
from jax.experimental.pallas import tpu as pltpu, tpu_sc as plsc

NUM_USER = 25000
NUM_ITEM = 25000
N_NODES = NUM_USER + NUM_ITEM
EMBED_DIM = 64
N_EDGES = 800000
N_LAYERS = 4

NC, NS = 2, 16
HALF = N_NODES // NC
ACC_ROWS = HALF + 8

NV = 64
RT_E = 12512
E_PAD = NV * RT_E
CHUNK = 128
REGION = 12544
R_CHUNKS = REGION // CHUNK
RBUF = REGION + 16
TRASH = REGION
BLK_CHUNKS = 14
BLK_PAIRS = BLK_CHUNKS // 2
BLK_E = BLK_CHUNKS * CHUNK

OUT_ROWS = N_NODES + 16
COPY_ROWS = 200
N_COPY = HALF // COPY_ROWS


def _route_body(src_hbm, dst_hbm, val_hbm,
                src_c, dloc_c, val_c, counts,
                inb_s, inb_d, inb_v, osrc, odloc, oval, cntv):
    core = lax.axis_index("c")
    tile = lax.axis_index("s")
    wid = core * NS + tile

    for p in range(NV // (NC * NS)):
        vt = wid + NC * NS * p
        in_base = vt * RT_E
        pltpu.sync_copy(src_hbm.at[pl.ds(in_base, RT_E)], inb_s)
        pltpu.sync_copy(dst_hbm.at[pl.ds(in_base, RT_E)], inb_d)
        pltpu.sync_copy(val_hbm.at[pl.ds(in_base, RT_E)], inb_v)

        zi = jnp.zeros((16,), jnp.int32)
        zf = jnp.zeros((16,), jnp.float32)

        def zero16(k, carry):
            sl = pl.ds(k * 16, 16)
            for h in range(NC):
                osrc[h][sl] = zi
                odloc[h][sl] = zi
                oval[h][sl] = zf
            return carry

        lax.fori_loop(0, RBUF // 16, zero16, 0)

        def group_body(g, ptrs):
            sl = pl.ds(g * 16, 16)
            s16 = inb_s[sl]
            d16 = inb_d[sl]
            v16 = inb_v[sl]
            lanes = lax.iota(jnp.int32, 16)
            new_ptrs = []
            for h in range(NC):
                dl16 = d16 - h * HALF
                m = (dl16 >= 0) & (dl16 < HALF)
                mi = jnp.where(m, jnp.ones((16,), jnp.int32),
                               jnp.zeros((16,), jnp.int32))
                v = mi
                for k in (1, 2, 4, 8):
                    idx = jnp.maximum(lanes - k, 0)
                    sh = v.at[idx].get(mode="promise_in_bounds")
                    v = v + jnp.where(lanes >= k, sh,
                                      jnp.zeros((16,), jnp.int32))
                cnt = v[15]
                ptr = ptrs[h]
                pos = jnp.where(m, ptr + (v - mi), TRASH)
                plsc.store_scatter(osrc[h], [pos], s16)
                plsc.store_scatter(odloc[h], [pos], dl16)
                plsc.store_scatter(oval[h], [pos], v16)
                new_ptrs.append(ptr + cnt)
            return tuple(new_ptrs)

        ptrs = lax.fori_loop(0, RT_E // 16, group_body,
                             (jnp.int32(0), jnp.int32(0)))

        for h in range(NC):
            region_off = (h * NV + vt) * REGION
            pltpu.sync_copy(osrc[h].at[pl.ds(0, REGION)],
                            src_c.at[pl.ds(region_off, REGION)])
            pltpu.sync_copy(odloc[h].at[pl.ds(0, REGION)],
                            dloc_c.at[pl.ds(region_off, REGION)])
            pltpu.sync_copy(oval[h].at[pl.ds(0, REGION)],
                            val_c.at[pl.ds(region_off, REGION)])
            cntv[pl.ds(0, 16)] = jnp.zeros((16,), jnp.int32) + ptrs[h]
            pltpu.sync_copy(cntv, counts.at[pl.ds((h * NV + vt) * 16, 16)])


def _layer_body(embeds_hbm, src_c, dloc2d, val_c, counts, zeros_hbm, out_hbm,
                acc, srcb, dlocb, valb, rows0, rows1, cntb, g0, g1, s0, s1, ib):
    core = lax.axis_index("c")
    tile = lax.axis_index("s")
    out_base = core * HALF

    for k in range(8):
        cidx = tile + NS * k
        @pl.when(cidx < N_COPY)
        def _():
            pltpu.sync_copy(zeros_hbm.at[pl.ds(cidx * COPY_ROWS, COPY_ROWS)],
                            acc.at[pl.ds(cidx * COPY_ROWS, COPY_ROWS)])
    plsc.subcore_barrier()

    dummy = zeros_hbm.at[pl.ds(0, CHUNK)]
    dummy_b = zeros_hbm.at[pl.ds(0, 3 * BLK_E // 64)]

    def drain(sem):
        pltpu.make_async_copy(dummy, rows0, sem).wait()

    def drain_blk():
        pltpu.make_async_copy(dummy_b, rows0.at[pl.ds(0, 3 * BLK_E // 64)],
                              ib).wait()

    def scale(rows, voff):
        for g in range(CHUNK // 16):
            v16 = valb[pl.ds(voff + g * 16, 16)]
            for e in range(16):
                ge = g * 16 + e
                v = v16[e]
                for j in range(EMBED_DIM // 16):
                    col = pl.ds(j * 16, 16)
                    rows[ge, col] = rows[ge, col] * v

    for p in range(4):
        vt = tile + NS * p
        rid = core * NV + vt
        region_off = rid * REGION

        pltpu.sync_copy(counts.at[pl.ds(rid * 16, 16)], cntb)
        cnt = cntb[pl.ds(0, 16)][0]
        npairs = lax.div(cnt + 2 * CHUNK - 1, 2 * CHUNK)
        nblocks = lax.div(npairs + BLK_PAIRS - 1, BLK_PAIRS)

        def load_block(b, par):
            base = region_off + b * BLK_E
            dst = pl.ds(par * BLK_E, BLK_E)
            pltpu.async_copy(src_c.at[pl.ds(base, BLK_E)], srcb.at[dst], ib)
            pltpu.async_copy(val_c.at[pl.ds(base, BLK_E)], valb.at[dst], ib)
            pltpu.async_copy(
                dloc2d.at[pl.ds(rid * R_CHUNKS + b * BLK_CHUNKS, BLK_CHUNKS)],
                dlocb.at[pl.ds(par * BLK_CHUNKS, BLK_CHUNKS)], ib)

        @pl.when(npairs > 0)
        def _():
            load_block(0, 0)
            drain_blk()
            pltpu.async_copy(embeds_hbm.at[srcb.at[pl.ds(0, CHUNK)]],
                             rows0, g0)

            def pair_body(i, carry):
                b = lax.div(i, BLK_PAIRS)
                lp = lax.rem(i, BLK_PAIRS)
                par = lax.rem(b, 2)
                o0 = par * BLK_E + lp * 2 * CHUNK
                o1 = o0 + CHUNK
                row0 = par * BLK_CHUNKS + lp * 2

                @pl.when(jnp.logical_and(lp == 0, i > 0))
                def _():
                    drain_blk()
                @pl.when(jnp.logical_and(lp == 0, b + 1 < nblocks))
                def _():
                    load_block(b + 1, 1 - par)

                @pl.when(i > 0)
                def _():
                    drain(s1)
                pltpu.async_copy(embeds_hbm.at[srcb.at[pl.ds(o1, CHUNK)]],
                                 rows1, g1)
                drain(g0)
                scale(rows0, o0)
                pltpu.async_copy(rows0, acc.at[dlocb.at[row0]], s0, add=True)
                drain(g1)
                drain(s0)

                @pl.when(i < npairs - 1)
                def _():
                    nb = lax.div(i + 1, BLK_PAIRS)
                    npar = lax.rem(nb, 2)
                    no0 = npar * BLK_E + lax.rem(i + 1, BLK_PAIRS) * 2 * CHUNK
                    pltpu.async_copy(embeds_hbm.at[srcb.at[pl.ds(no0, CHUNK)]],
                                     rows0, g0)
                scale(rows1, o1)
                pltpu.async_copy(rows1, acc.at[dlocb.at[row0 + 1]], s1,
                                 add=True)
                return carry

            lax.fori_loop(0, npairs, pair_body, 0)
            drain(s1)

    plsc.subcore_barrier()

    for k in range(8):
        cidx = tile + NS * k
        @pl.when(cidx < N_COPY)
        def _():
            pltpu.sync_copy(
                acc.at[pl.ds(cidx * COPY_ROWS, COPY_ROWS)],
                out_hbm.at[pl.ds(out_base + cidx * COPY_ROWS, COPY_ROWS)])


@jax.jit
def _route(src, dst, vals):
    mesh = plsc.VectorSubcoreMesh(core_axis_name="c", subcore_axis_name="s")
    n = NC * NV * REGION
    f = pl.kernel(
        _route_body,
        out_type=(
            jax.ShapeDtypeStruct((n,), jnp.int32),
            jax.ShapeDtypeStruct((n,), jnp.int32),
            jax.ShapeDtypeStruct((n,), jnp.float32),
            jax.ShapeDtypeStruct((NC * NV * 16,), jnp.int32),
        ),
        mesh=mesh,
        scratch_types=[
            pltpu.VMEM((RT_E,), jnp.int32),
            pltpu.VMEM((RT_E,), jnp.int32),
            pltpu.VMEM((RT_E,), jnp.float32),
            [pltpu.VMEM((RBUF,), jnp.int32) for _ in range(NC)],
            [pltpu.VMEM((RBUF,), jnp.int32) for _ in range(NC)],
            [pltpu.VMEM((RBUF,), jnp.float32) for _ in range(NC)],
            pltpu.VMEM((16,), jnp.int32),
        ],
        compiler_params=pltpu.CompilerParams(use_tc_tiling_on_sc=False,
                                             needs_layout_passes=False),
    )
    return f(src, dst, vals)


@jax.jit
def _run_layer(embeds, src_c, dloc2d, val_c, counts, zeros):
    mesh = plsc.VectorSubcoreMesh(core_axis_name="c", subcore_axis_name="s")
    f = pl.kernel(
        _layer_body,
        out_type=jax.ShapeDtypeStruct((OUT_ROWS, EMBED_DIM), jnp.float32),
        mesh=mesh,
        scratch_types=[
            pltpu.VMEM_SHARED((ACC_ROWS, EMBED_DIM), jnp.float32),
            pltpu.VMEM((2 * BLK_E,), jnp.int32),
            pltpu.VMEM((2 * BLK_CHUNKS, CHUNK), jnp.int32),
            pltpu.VMEM((2 * BLK_E,), jnp.float32),
            pltpu.VMEM((CHUNK, EMBED_DIM), jnp.float32),
            pltpu.VMEM((CHUNK, EMBED_DIM), jnp.float32),
            pltpu.VMEM((16,), jnp.int32),
            pltpu.SemaphoreType.DMA,
            pltpu.SemaphoreType.DMA,
            pltpu.SemaphoreType.DMA,
            pltpu.SemaphoreType.DMA,
            pltpu.SemaphoreType.DMA,
        ],
        compiler_params=pltpu.CompilerParams(use_tc_tiling_on_sc=False,
                                             needs_layout_passes=False),
    )
    return f(embeds, src_c, dloc2d, val_c, counts, zeros)


def kernel(user_emb, item_emb, edge_vals, edge_index):
    embeds = jnp.concatenate([user_emb, item_emb], axis=0)
    embeds = jnp.pad(embeds, ((0, OUT_ROWS - N_NODES), (0, 0)))
    pad_e = E_PAD - N_EDGES
    src = jnp.pad(edge_index[0], (0, pad_e))
    dst = jnp.pad(edge_index[1], (0, pad_e), constant_values=N_NODES + 1)
    vals = jnp.pad(edge_vals, (0, pad_e))
    zeros = jnp.zeros((HALF, EMBED_DIM), jnp.float32)

    src_c, dloc_c, val_c, counts = _route(src, dst, vals)
    dloc2d = dloc_c.reshape(NC * NV * R_CHUNKS, CHUNK)
    for _ in range(N_LAYERS):
        embeds = _run_layer(embeds, src_c, dloc2d, val_c, counts, zeros)
    return embeds[:NUM_USER], embeds[NUM_USER:N_NODES]
